# Initial kernel scaffold; baseline (speedup 1.0000x reference)
import jax
import jax.numpy as jnp
from jax.experimental import pallas as pl


def kernel(fatoms, fbonds, W_i, W_h, W_o, agraph, bgraph):
    raise NotImplementedError("write your pallas kernel here")


# speedup vs baseline: 2.9077x; 2.9077x over previous
import jax
import jax.numpy as jnp
from jax import lax
from jax.experimental import pallas as pl
from jax.experimental.pallas import tpu as pltpu
from jax.experimental.pallas import tpu_sc as plsc

HIDDEN = 128
MAX_NB = 16
N_ATOMS = 10000
N_BONDS = 160000
N_MOLS = 100
ATOMS_PER_MOL = N_ATOMS // N_MOLS
N_ATOMS_PAD = 10240

NC, NS = 2, 16
NW = NC * NS
CHUNK = 128


def _make_gather_sum(n_rows_out, name):
  assert n_rows_out % CHUNK == 0
  total_chunks = n_rows_out // CHUNK
  n_iter = -(-total_chunks // NW)
  mesh = plsc.VectorSubcoreMesh(
      core_axis_name="c", subcore_axis_name="s", num_cores=NC, num_subcores=NS
  )

  def body(table_hbm, idxt_hbm, out_hbm, idx_v, acc_v, sem):
    wid = lax.axis_index("s") * NC + lax.axis_index("c")

    def chunk_body(i, carry):
      c = i * NW + wid

      @pl.when(c < total_chunks)
      def _():
        base = c * CHUNK
        pltpu.sync_copy(idxt_hbm.at[:, pl.ds(base, CHUNK)], idx_v)
        pltpu.async_copy(table_hbm.at[idx_v.at[0]], acc_v, sem).wait()
        cps = [
            pltpu.async_copy(table_hbm.at[idx_v.at[k]], acc_v, sem, add=True)
            for k in range(1, MAX_NB)
        ]
        for cp in cps:
          cp.wait()
        pltpu.sync_copy(acc_v, out_hbm.at[pl.ds(base, CHUNK)])

      return carry

    lax.fori_loop(0, n_iter, chunk_body, 0)

  return pl.kernel(
      body,
      out_type=jax.ShapeDtypeStruct((n_rows_out, HIDDEN), jnp.float32),
      mesh=mesh,
      scratch_types=[
          pltpu.VMEM((MAX_NB, CHUNK), jnp.int32),
          pltpu.VMEM((CHUNK, HIDDEN), jnp.float32),
          pltpu.SemaphoreType.DMA,
      ],
      name=name,
  )


_gather_cache = {}


def _gather_sum(n_rows_out, name):
  key = (n_rows_out, name)
  if key not in _gather_cache:
    _gather_cache[key] = _make_gather_sum(n_rows_out, name)
  return _gather_cache[key]


_MM_ROWS = 4000


def _binput_body(fb_ref, wi_ref, bi_ref, msg_ref):
  bi = jnp.dot(fb_ref[...], wi_ref[...], preferred_element_type=jnp.float32)
  bi_ref[...] = bi
  msg_ref[...] = jnp.maximum(bi, 0.0)


_binput_mm = pl.pallas_call(
    _binput_body,
    grid=(N_BONDS // _MM_ROWS,),
    in_specs=[
        pl.BlockSpec((_MM_ROWS, HIDDEN), lambda i: (i, 0)),
        pl.BlockSpec((HIDDEN, HIDDEN), lambda i: (0, 0)),
    ],
    out_specs=[
        pl.BlockSpec((_MM_ROWS, HIDDEN), lambda i: (i, 0)),
        pl.BlockSpec((_MM_ROWS, HIDDEN), lambda i: (i, 0)),
    ],
    out_shape=[
        jax.ShapeDtypeStruct((N_BONDS, HIDDEN), jnp.float32),
        jax.ShapeDtypeStruct((N_BONDS, HIDDEN), jnp.float32),
    ],
)


def _round_body(bi_ref, nei_ref, wh_ref, out_ref):
  acc = jnp.dot(nei_ref[...], wh_ref[...], preferred_element_type=jnp.float32)
  out_ref[...] = jnp.maximum(bi_ref[...] + acc, 0.0)


_round_mm = pl.pallas_call(
    _round_body,
    grid=(N_BONDS // _MM_ROWS,),
    in_specs=[
        pl.BlockSpec((_MM_ROWS, HIDDEN), lambda i: (i, 0)),
        pl.BlockSpec((_MM_ROWS, HIDDEN), lambda i: (i, 0)),
        pl.BlockSpec((HIDDEN, HIDDEN), lambda i: (0, 0)),
    ],
    out_specs=pl.BlockSpec((_MM_ROWS, HIDDEN), lambda i: (i, 0)),
    out_shape=jax.ShapeDtypeStruct((N_BONDS, HIDDEN), jnp.float32),
)


def _final_body(fa_ref, an_ref, wo1_ref, wo2_ref, out_ref):
  h = jnp.dot(fa_ref[...], wo1_ref[...], preferred_element_type=jnp.float32)
  h += jnp.dot(an_ref[...], wo2_ref[...], preferred_element_type=jnp.float32)
  h = jnp.maximum(h, 0.0)
  rows = lax.broadcasted_iota(jnp.int32, (N_MOLS, N_ATOMS), 0)
  cols = lax.broadcasted_iota(jnp.int32, (N_MOLS, N_ATOMS), 1)
  pool = jnp.where(cols // ATOMS_PER_MOL == rows, 1.0 / ATOMS_PER_MOL, 0.0)
  out_ref[...] = jnp.dot(pool, h, preferred_element_type=jnp.float32)


_final_mm = pl.pallas_call(
    _final_body,
    grid=(1,),
    in_specs=[
        pl.BlockSpec((N_ATOMS, HIDDEN), lambda i: (0, 0)),
        pl.BlockSpec((N_ATOMS, HIDDEN), lambda i: (0, 0)),
        pl.BlockSpec((HIDDEN, HIDDEN), lambda i: (0, 0)),
        pl.BlockSpec((HIDDEN, HIDDEN), lambda i: (0, 0)),
    ],
    out_specs=pl.BlockSpec((N_MOLS, HIDDEN), lambda i: (0, 0)),
    out_shape=jax.ShapeDtypeStruct((N_MOLS, HIDDEN), jnp.float32),
)


def kernel(fatoms, fbonds, W_i, W_h, W_o, agraph, bgraph):
  bgraph_t = bgraph.T
  agraph_t = jnp.pad(agraph, ((0, N_ATOMS_PAD - N_ATOMS), (0, 0))).T
  binput, message = _binput_mm(fbonds, W_i)
  for _ in range(2):
    nei = _gather_sum(N_BONDS, "sc_gather_bonds")(message, bgraph_t)
    message = _round_mm(binput, nei, W_h)
  anei = _gather_sum(N_ATOMS_PAD, "sc_gather_atoms")(message, agraph_t)[:N_ATOMS]
  return _final_mm(fatoms, anei, W_o[:HIDDEN], W_o[HIDDEN:])


# speedup vs baseline: 3.0677x; 1.0550x over previous
import jax
import jax.numpy as jnp
from jax import lax
from jax.experimental import pallas as pl
from jax.experimental.pallas import tpu as pltpu
from jax.experimental.pallas import tpu_sc as plsc

HIDDEN = 128
MAX_NB = 16
N_ATOMS = 10000
N_BONDS = 160000
N_MOLS = 100
ATOMS_PER_MOL = N_ATOMS // N_MOLS
N_ATOMS_PAD = 10240

NC, NS = 2, 16
NW = NC * NS
CHUNK = 128


def _make_gather_sum(n_rows_out, name):
  assert n_rows_out % CHUNK == 0
  total_chunks = n_rows_out // CHUNK
  n_iter = -(-total_chunks // NW)
  mesh = plsc.VectorSubcoreMesh(
      core_axis_name="c", subcore_axis_name="s", num_cores=NC, num_subcores=NS
  )

  n_outer = -(-n_iter // 2)

  def body(table_hbm, idxt_hbm, out_hbm, idx_v, acc_v, gsem, osem):
    wid = lax.axis_index("s") * NC + lax.axis_index("c")

    def outer_body(j, carry):
      for b in range(2):
        c = (2 * j + b) * NW + wid

        @pl.when(jnp.logical_and(j > 0, c - 2 * NW < total_chunks))
        def _(b=b):
          pltpu.make_async_copy(
              out_hbm.at[pl.ds(0, CHUNK)], acc_v.at[b], osem[b]
          ).wait()

        @pl.when(c < total_chunks)
        def _(b=b, c=c):
          base = c * CHUNK
          pltpu.sync_copy(idxt_hbm.at[:, pl.ds(base, CHUNK)], idx_v.at[b])
          pltpu.async_copy(
              table_hbm.at[idx_v.at[b, 0]], acc_v.at[b], gsem[b]
          ).wait()
          for k in range(1, MAX_NB):
            pltpu.async_copy(
                table_hbm.at[idx_v.at[b, k]], acc_v.at[b], gsem[b], add=True
            )

      for b in range(2):
        c = (2 * j + b) * NW + wid

        @pl.when(c < total_chunks)
        def _(b=b, c=c):
          for _k in range(1, MAX_NB):
            pltpu.make_async_copy(
                table_hbm.at[idx_v.at[b, 0]], acc_v.at[b], gsem[b]
            ).wait()
          pltpu.async_copy(acc_v.at[b], out_hbm.at[pl.ds(c * CHUNK, CHUNK)],
                           osem[b])

      return carry

    lax.fori_loop(0, n_outer, outer_body, 0)

    for b in range(2):
      c_last = (2 * (n_outer - 1) + b) * NW + wid

      @pl.when(c_last < total_chunks)
      def _(b=b):
        pltpu.make_async_copy(
            out_hbm.at[pl.ds(0, CHUNK)], acc_v.at[b], osem[b]
        ).wait()

  return pl.kernel(
      body,
      out_type=jax.ShapeDtypeStruct((n_rows_out, HIDDEN), jnp.float32),
      mesh=mesh,
      scratch_types=[
          pltpu.VMEM((2, MAX_NB, CHUNK), jnp.int32),
          pltpu.VMEM((2, CHUNK, HIDDEN), jnp.float32),
          [pltpu.SemaphoreType.DMA] * 2,
          [pltpu.SemaphoreType.DMA] * 2,
      ],
      name=name,
  )


_gather_cache = {}


def _gather_sum(n_rows_out, name):
  key = (n_rows_out, name)
  if key not in _gather_cache:
    _gather_cache[key] = _make_gather_sum(n_rows_out, name)
  return _gather_cache[key]


_MM_ROWS = 4000


def _binput_body(fb_ref, wi_ref, bi_ref, msg_ref):
  bi = jnp.dot(fb_ref[...], wi_ref[...], preferred_element_type=jnp.float32)
  bi_ref[...] = bi
  msg_ref[...] = jnp.maximum(bi, 0.0)


_binput_mm = pl.pallas_call(
    _binput_body,
    grid=(N_BONDS // _MM_ROWS,),
    in_specs=[
        pl.BlockSpec((_MM_ROWS, HIDDEN), lambda i: (i, 0)),
        pl.BlockSpec((HIDDEN, HIDDEN), lambda i: (0, 0)),
    ],
    out_specs=[
        pl.BlockSpec((_MM_ROWS, HIDDEN), lambda i: (i, 0)),
        pl.BlockSpec((_MM_ROWS, HIDDEN), lambda i: (i, 0)),
    ],
    out_shape=[
        jax.ShapeDtypeStruct((N_BONDS, HIDDEN), jnp.float32),
        jax.ShapeDtypeStruct((N_BONDS, HIDDEN), jnp.float32),
    ],
)


def _round_body(bi_ref, nei_ref, wh_ref, out_ref):
  acc = jnp.dot(nei_ref[...], wh_ref[...], preferred_element_type=jnp.float32)
  out_ref[...] = jnp.maximum(bi_ref[...] + acc, 0.0)


_round_mm = pl.pallas_call(
    _round_body,
    grid=(N_BONDS // _MM_ROWS,),
    in_specs=[
        pl.BlockSpec((_MM_ROWS, HIDDEN), lambda i: (i, 0)),
        pl.BlockSpec((_MM_ROWS, HIDDEN), lambda i: (i, 0)),
        pl.BlockSpec((HIDDEN, HIDDEN), lambda i: (0, 0)),
    ],
    out_specs=pl.BlockSpec((_MM_ROWS, HIDDEN), lambda i: (i, 0)),
    out_shape=jax.ShapeDtypeStruct((N_BONDS, HIDDEN), jnp.float32),
)


def _final_body(fa_ref, an_ref, wo1_ref, wo2_ref, out_ref):
  h = jnp.dot(fa_ref[...], wo1_ref[...], preferred_element_type=jnp.float32)
  h += jnp.dot(an_ref[...], wo2_ref[...], preferred_element_type=jnp.float32)
  h = jnp.maximum(h, 0.0)
  rows = lax.broadcasted_iota(jnp.int32, (N_MOLS, N_ATOMS), 0)
  cols = lax.broadcasted_iota(jnp.int32, (N_MOLS, N_ATOMS), 1)
  pool = jnp.where(cols // ATOMS_PER_MOL == rows, 1.0 / ATOMS_PER_MOL, 0.0)
  out_ref[...] = jnp.dot(pool, h, preferred_element_type=jnp.float32)


_final_mm = pl.pallas_call(
    _final_body,
    grid=(1,),
    in_specs=[
        pl.BlockSpec((N_ATOMS, HIDDEN), lambda i: (0, 0)),
        pl.BlockSpec((N_ATOMS, HIDDEN), lambda i: (0, 0)),
        pl.BlockSpec((HIDDEN, HIDDEN), lambda i: (0, 0)),
        pl.BlockSpec((HIDDEN, HIDDEN), lambda i: (0, 0)),
    ],
    out_specs=pl.BlockSpec((N_MOLS, HIDDEN), lambda i: (0, 0)),
    out_shape=jax.ShapeDtypeStruct((N_MOLS, HIDDEN), jnp.float32),
)


def kernel(fatoms, fbonds, W_i, W_h, W_o, agraph, bgraph):
  bgraph_t = bgraph.T
  agraph_t = jnp.pad(agraph, ((0, N_ATOMS_PAD - N_ATOMS), (0, 0))).T
  binput, message = _binput_mm(fbonds, W_i)
  for _ in range(2):
    nei = _gather_sum(N_BONDS, "sc_gather_bonds")(message, bgraph_t)
    message = _round_mm(binput, nei, W_h)
  anei = _gather_sum(N_ATOMS_PAD, "sc_gather_atoms")(message, agraph_t)[:N_ATOMS]
  return _final_mm(fatoms, anei, W_o[:HIDDEN], W_o[HIDDEN:])


# speedup vs baseline: 3.1711x; 1.0337x over previous
import jax
import jax.numpy as jnp
from jax import lax
from jax.experimental import pallas as pl
from jax.experimental.pallas import tpu as pltpu
from jax.experimental.pallas import tpu_sc as plsc

HIDDEN = 128
MAX_NB = 16
N_ATOMS = 10000
N_BONDS = 160000
N_MOLS = 100
ATOMS_PER_MOL = N_ATOMS // N_MOLS
N_ATOMS_PAD = 10240

NC, NS = 2, 16
NW = NC * NS
CHUNK = 128
NBUF = 4


def _make_gather_sum(n_rows_out, name):
  assert n_rows_out % CHUNK == 0
  total_chunks = n_rows_out // CHUNK
  n_iter = -(-total_chunks // NW)
  mesh = plsc.VectorSubcoreMesh(
      core_axis_name="c", subcore_axis_name="s", num_cores=NC, num_subcores=NS
  )

  n_outer = -(-n_iter // NBUF)

  def body(table_hbm, idxt_hbm, out_hbm, idx_v, acc_v, gsem, osem):
    wid = lax.axis_index("s") * NC + lax.axis_index("c")

    def outer_body(j, carry):
      for b in range(NBUF):
        c = (NBUF * j + b) * NW + wid

        @pl.when(jnp.logical_and(j > 0, c - NBUF * NW < total_chunks))
        def _(b=b):
          pltpu.make_async_copy(
              out_hbm.at[pl.ds(0, CHUNK)], acc_v.at[b], osem[b]
          ).wait()

        @pl.when(c < total_chunks)
        def _(b=b, c=c):
          base = c * CHUNK
          pltpu.sync_copy(idxt_hbm.at[:, pl.ds(base, CHUNK)], idx_v.at[b])
          pltpu.async_copy(
              table_hbm.at[idx_v.at[b, 0]], acc_v.at[b], gsem[b]
          ).wait()
          for k in range(1, MAX_NB):
            pltpu.async_copy(
                table_hbm.at[idx_v.at[b, k]], acc_v.at[b], gsem[b], add=True
            )

      for b in range(NBUF):
        c = (NBUF * j + b) * NW + wid

        @pl.when(c < total_chunks)
        def _(b=b, c=c):
          for _k in range(1, MAX_NB):
            pltpu.make_async_copy(
                table_hbm.at[idx_v.at[b, 0]], acc_v.at[b], gsem[b]
            ).wait()
          pltpu.async_copy(acc_v.at[b], out_hbm.at[pl.ds(c * CHUNK, CHUNK)],
                           osem[b])

      return carry

    lax.fori_loop(0, n_outer, outer_body, 0)

    for b in range(NBUF):
      c_last = (NBUF * (n_outer - 1) + b) * NW + wid

      @pl.when(c_last < total_chunks)
      def _(b=b):
        pltpu.make_async_copy(
            out_hbm.at[pl.ds(0, CHUNK)], acc_v.at[b], osem[b]
        ).wait()

  return pl.kernel(
      body,
      out_type=jax.ShapeDtypeStruct((n_rows_out, HIDDEN), jnp.float32),
      mesh=mesh,
      scratch_types=[
          pltpu.VMEM((NBUF, MAX_NB, CHUNK), jnp.int32),
          pltpu.VMEM((NBUF, CHUNK, HIDDEN), jnp.float32),
          [pltpu.SemaphoreType.DMA] * NBUF,
          [pltpu.SemaphoreType.DMA] * NBUF,
      ],
      name=name,
  )


_gather_cache = {}


def _gather_sum(n_rows_out, name):
  key = (n_rows_out, name)
  if key not in _gather_cache:
    _gather_cache[key] = _make_gather_sum(n_rows_out, name)
  return _gather_cache[key]


_MM_ROWS = 4000


def _binput_body(fb_ref, wi_ref, bi_ref, msg_ref):
  bi = jnp.dot(fb_ref[...], wi_ref[...], preferred_element_type=jnp.float32)
  bi_ref[...] = bi
  msg_ref[...] = jnp.maximum(bi, 0.0)


_binput_mm = pl.pallas_call(
    _binput_body,
    grid=(N_BONDS // _MM_ROWS,),
    in_specs=[
        pl.BlockSpec((_MM_ROWS, HIDDEN), lambda i: (i, 0)),
        pl.BlockSpec((HIDDEN, HIDDEN), lambda i: (0, 0)),
    ],
    out_specs=[
        pl.BlockSpec((_MM_ROWS, HIDDEN), lambda i: (i, 0)),
        pl.BlockSpec((_MM_ROWS, HIDDEN), lambda i: (i, 0)),
    ],
    out_shape=[
        jax.ShapeDtypeStruct((N_BONDS, HIDDEN), jnp.float32),
        jax.ShapeDtypeStruct((N_BONDS, HIDDEN), jnp.float32),
    ],
)


def _round_body(bi_ref, nei_ref, wh_ref, out_ref):
  acc = jnp.dot(nei_ref[...], wh_ref[...], preferred_element_type=jnp.float32)
  out_ref[...] = jnp.maximum(bi_ref[...] + acc, 0.0)


_round_mm = pl.pallas_call(
    _round_body,
    grid=(N_BONDS // _MM_ROWS,),
    in_specs=[
        pl.BlockSpec((_MM_ROWS, HIDDEN), lambda i: (i, 0)),
        pl.BlockSpec((_MM_ROWS, HIDDEN), lambda i: (i, 0)),
        pl.BlockSpec((HIDDEN, HIDDEN), lambda i: (0, 0)),
    ],
    out_specs=pl.BlockSpec((_MM_ROWS, HIDDEN), lambda i: (i, 0)),
    out_shape=jax.ShapeDtypeStruct((N_BONDS, HIDDEN), jnp.float32),
)


def _final_body(fa_ref, an_ref, wo1_ref, wo2_ref, out_ref):
  h = jnp.dot(fa_ref[...], wo1_ref[...], preferred_element_type=jnp.float32)
  h += jnp.dot(an_ref[...], wo2_ref[...], preferred_element_type=jnp.float32)
  h = jnp.maximum(h, 0.0)
  rows = lax.broadcasted_iota(jnp.int32, (N_MOLS, N_ATOMS), 0)
  cols = lax.broadcasted_iota(jnp.int32, (N_MOLS, N_ATOMS), 1)
  pool = jnp.where(cols // ATOMS_PER_MOL == rows, 1.0 / ATOMS_PER_MOL, 0.0)
  out_ref[...] = jnp.dot(pool, h, preferred_element_type=jnp.float32)


_final_mm = pl.pallas_call(
    _final_body,
    grid=(1,),
    in_specs=[
        pl.BlockSpec((N_ATOMS, HIDDEN), lambda i: (0, 0)),
        pl.BlockSpec((N_ATOMS, HIDDEN), lambda i: (0, 0)),
        pl.BlockSpec((HIDDEN, HIDDEN), lambda i: (0, 0)),
        pl.BlockSpec((HIDDEN, HIDDEN), lambda i: (0, 0)),
    ],
    out_specs=pl.BlockSpec((N_MOLS, HIDDEN), lambda i: (0, 0)),
    out_shape=jax.ShapeDtypeStruct((N_MOLS, HIDDEN), jnp.float32),
)


def kernel(fatoms, fbonds, W_i, W_h, W_o, agraph, bgraph):
  bgraph_t = bgraph.T
  agraph_t = jnp.pad(agraph, ((0, N_ATOMS_PAD - N_ATOMS), (0, 0))).T
  binput, message = _binput_mm(fbonds, W_i)
  for _ in range(2):
    nei = _gather_sum(N_BONDS, "sc_gather_bonds")(message, bgraph_t)
    message = _round_mm(binput, nei, W_h)
  anei = _gather_sum(N_ATOMS_PAD, "sc_gather_atoms")(message, agraph_t)[:N_ATOMS]
  return _final_mm(fatoms, anei, W_o[:HIDDEN], W_o[HIDDEN:])
